# Initial kernel scaffold; baseline (speedup 1.0000x reference)
import jax
import jax.numpy as jnp
from jax.experimental import pallas as pl


def kernel(scores, bbox_deltas, im_info):
    raise NotImplementedError("write your pallas kernel here")


# speedup vs baseline: 231.1614x; 231.1614x over previous
import functools

import jax
import jax.numpy as jnp
import numpy as np
from jax.experimental import pallas as pl

_ANCHOR_BASES = np.array(
    [[-84.0, -40.0, 99.0, 55.0], [-176.0, -88.0, 191.0, 103.0],
     [-360.0, -184.0, 375.0, 199.0], [-56.0, -56.0, 71.0, 71.0],
     [-120.0, -120.0, 135.0, 135.0], [-248.0, -248.0, 263.0, 263.0],
     [-36.0, -80.0, 51.0, 95.0], [-80.0, -168.0, 95.0, 183.0],
     [-168.0, -344.0, 183.0, 359.0]], dtype=np.float32)
_STRIDE = 16
_PRE_NMS_TOP_N = 6000
_POST_NMS_TOP_N = 300
_NMS_THRESH = 0.7

_H = _W = 48
_A = 9
_N = _H * _W * _A
_ROWS = 168
_NPAD = _ROWS * 128


def _np_anchors():
    shift_x = np.arange(0, _W) * _STRIDE
    shift_y = np.arange(0, _H) * _STRIDE
    sx, sy = np.meshgrid(shift_x, shift_y)
    shifts = np.stack([sx.ravel(), sy.ravel(), sx.ravel(), sy.ravel()],
                      axis=1).astype(np.float32)
    anchors = _ANCHOR_BASES.reshape(1, _A, 4) + shifts.reshape(-1, 1, 4)
    return anchors.reshape(_N, 4)


_ANCHORS_NP = _np_anchors()

def _anchor_consts():
    a = _ANCHORS_NP
    widths = a[:, 2] - a[:, 0] + 1.0
    heights = a[:, 3] - a[:, 1] + 1.0
    ctr_x = a[:, 0] + 0.5 * widths
    ctr_y = a[:, 1] + 0.5 * heights
    out = np.zeros((4, _NPAD), dtype=np.float32)
    out[0, :_N] = widths
    out[1, :_N] = heights
    out[0, _N:] = 1.0
    out[1, _N:] = 1.0
    out[2, :_N] = ctr_x
    out[3, :_N] = ctr_y
    return out.reshape(4, _ROWS, 128)


_ANC4_NP = _anchor_consts()


def _rpn_kernel(sc_ref, dx_ref, dy_ref, dw_ref, dh_ref, anc_ref,
                hm_ref, wm_ref, so_ref, bo_ref):
    f32 = jnp.float32
    B = sc_ref.shape[0]
    scv = sc_ref[...]
    dx = dx_ref[...]
    dy = dy_ref[...]
    dw = dw_ref[...]
    dh = dh_ref[...]
    WA = anc_ref[0][None]
    HA = anc_ref[1][None]
    CX = anc_ref[2][None]
    CY = anc_ref[3][None]
    hm = jnp.max(hm_ref[...], axis=(1, 2), keepdims=True)
    wm = jnp.max(wm_ref[...], axis=(1, 2), keepdims=True)

    pcx = dx * WA + CX
    pcy = dy * HA + CY
    pw = jnp.exp(dw) * WA
    ph = jnp.exp(dh) * HA
    x1 = jnp.minimum(jnp.maximum(pcx - 0.5 * pw, 0.0), wm)
    y1 = jnp.minimum(jnp.maximum(pcy - 0.5 * ph, 0.0), hm)
    x2 = jnp.minimum(jnp.maximum(pcx + 0.5 * pw, 0.0), wm)
    y2 = jnp.minimum(jnp.maximum(pcy + 0.5 * ph, 0.0), hm)
    areas = (x2 - x1 + 1.0) * (y2 - y1 + 1.0)

    lin = (jax.lax.broadcasted_iota(jnp.int32, (B, _ROWS, 128), 1) * 128
           + jax.lax.broadcasted_iota(jnp.int32, (B, _ROWS, 128), 2))

    bits = jax.lax.bitcast_convert_type(scv, jnp.int32)
    K = _PRE_NMS_TOP_N

    def sel_body(t, pfx):
        cand = pfx | (jnp.int32(1) << (jnp.int32(30) - t))
        cnt = jnp.sum((bits >= cand).astype(jnp.int32), axis=(1, 2),
                      keepdims=True)
        return jnp.where(cnt >= K, cand, pfx)

    v = jax.lax.fori_loop(0, 31, sel_body,
                          jnp.zeros((B, 1, 1), jnp.int32))

    gt = bits > v
    eq = bits == v
    cnt_gt = jnp.sum(gt.astype(jnp.int32), axis=(1, 2), keepdims=True)
    m = (K - cnt_gt).astype(f32)

    r0 = jax.lax.broadcasted_iota(jnp.int32, (_ROWS, _ROWS), 0)
    r1 = jax.lax.broadcasted_iota(jnp.int32, (_ROWS, _ROWS), 1)
    TL = (r1 < r0).astype(f32)
    c0 = jax.lax.broadcasted_iota(jnp.int32, (128, 128), 0)
    c1 = jax.lax.broadcasted_iota(jnp.int32, (128, 128), 1)
    MU = (c0 < c1).astype(f32)
    eqf = eq.astype(f32)
    pcs = []
    for i in range(B):
        e = eqf[i]
        rowp = jnp.sum(
            jax.lax.dot(TL, e, preferred_element_type=f32),
            axis=1, keepdims=True)
        lanep = jax.lax.dot(e, MU, preferred_element_type=f32)
        pcs.append((rowp + lanep)[None])
    pc = jnp.concatenate(pcs, axis=0)
    alive0 = (gt | (eq & (pc < m))).astype(f32)

    i8 = jax.lax.broadcasted_iota(jnp.int32, (8, 128), 0)
    i128 = jax.lax.broadcasted_iota(jnp.int32, (8, 128), 1)
    img_id = jax.lax.broadcasted_iota(jnp.int32, (B, 1, 1), 0).astype(f32)
    BIG = jnp.int32(2 ** 30)

    def body(r, carry):
        alive, sa, xa, ya, x2a, y2a = carry
        alive_b = alive > 0.0
        ms = jnp.where(alive_b, scv, -1.0)
        mx = jnp.max(ms, axis=(1, 2), keepdims=True)
        validr = mx >= 0.0
        hit = (ms == mx) & alive_b
        idx = jnp.min(jnp.where(hit, lin, BIG), axis=(1, 2), keepdims=True)
        sel = hit & (lin == idx)
        sm = sel.astype(f32)
        bx1 = jnp.sum(sm * x1, axis=(1, 2), keepdims=True)
        by1 = jnp.sum(sm * y1, axis=(1, 2), keepdims=True)
        bx2 = jnp.sum(sm * x2, axis=(1, 2), keepdims=True)
        by2 = jnp.sum(sm * y2, axis=(1, 2), keepdims=True)
        barea = (bx2 - bx1 + 1.0) * (by2 - by1 + 1.0)
        xx1 = jnp.maximum(bx1, x1)
        yy1 = jnp.maximum(by1, y1)
        xx2 = jnp.minimum(bx2, x2)
        yy2 = jnp.minimum(by2, y2)
        iw = jnp.maximum(0.0, xx2 - xx1 + 1.0)
        ih = jnp.maximum(0.0, yy2 - yy1 + 1.0)
        inter = iw * ih
        iou = inter / (barea + areas - inter)
        alive = jnp.where(iou > _NMS_THRESH, 0.0, alive)
        wmask = ((i8 == (r // 128)) & (i128 == (r % 128)))[None]
        sval = jnp.where(validr, mx, img_id)
        sa = jnp.where(wmask, sval, sa)
        xa = jnp.where(wmask, jnp.where(validr, bx1, 0.0), xa)
        ya = jnp.where(wmask, jnp.where(validr, by1, 0.0), ya)
        x2a = jnp.where(wmask, jnp.where(validr, bx2, 0.0), x2a)
        y2a = jnp.where(wmask, jnp.where(validr, by2, 0.0), y2a)
        return alive, sa, xa, ya, x2a, y2a

    z = jnp.zeros((B, 8, 128), f32)
    _, sa, xa, ya, x2a, y2a = jax.lax.fori_loop(
        0, _POST_NMS_TOP_N, body, (alive0, z, z, z, z, z))
    so_ref[...] = sa
    bo_ref[:, 0] = xa
    bo_ref[:, 1] = ya
    bo_ref[:, 2] = x2a
    bo_ref[:, 3] = y2a


@functools.partial(jax.jit, static_argnames=())
def kernel(scores, bbox_deltas, im_info):
    f32 = jnp.float32
    B = scores.shape[0]
    fg = scores[:, _A:, :, :]
    sc = jnp.transpose(fg, (0, 2, 3, 1)).reshape(B, _N)
    deltas = jnp.transpose(bbox_deltas, (0, 2, 3, 1)).reshape(B, _N, 4)
    scp = jnp.pad(sc, ((0, 0), (0, _NPAD - _N)),
                  constant_values=-1.0).reshape(B, _ROWS, 128)
    dpad = jnp.pad(deltas, ((0, 0), (0, _NPAD - _N), (0, 0)))
    dxp = dpad[..., 0].reshape(B, _ROWS, 128)
    dyp = dpad[..., 1].reshape(B, _ROWS, 128)
    dwp = dpad[..., 2].reshape(B, _ROWS, 128)
    dhp = dpad[..., 3].reshape(B, _ROWS, 128)
    anc4 = jnp.asarray(_ANC4_NP)
    hmb = jnp.broadcast_to((im_info[:, 0] - 1.0)[:, None, None], (B, 8, 128))
    wmb = jnp.broadcast_to((im_info[:, 1] - 1.0)[:, None, None], (B, 8, 128))

    so, bo = pl.pallas_call(
        _rpn_kernel,
        out_shape=[
            jax.ShapeDtypeStruct((B, 8, 128), f32),
            jax.ShapeDtypeStruct((B, 4, 8, 128), f32),
        ],
    )(scp, dxp, dyp, dwp, dhp, anc4, hmb, wmb)

    s = so.reshape(B, 8 * 128)[:, :_POST_NMS_TOP_N][..., None]
    b = jnp.transpose(bo.reshape(B, 4, 8 * 128)[:, :, :_POST_NMS_TOP_N],
                      (0, 2, 1))
    bcol = jnp.broadcast_to(
        jnp.arange(B, dtype=f32)[:, None, None], (B, _POST_NMS_TOP_N, 1))
    rpn_bbox = jnp.concatenate([bcol, b], axis=2)
    anchors = jnp.asarray(_ANCHORS_NP)
    return s, rpn_bbox, anchors
